# Initial kernel scaffold; baseline (speedup 1.0000x reference)
import functools

import jax
import jax.numpy as jnp
from jax import lax
from jax.experimental import pallas as pl
from jax.experimental.pallas import tpu as pltpu
from jax.experimental.pallas import tpu_sc as plsc

N = 320000
D = 128
S = 10000
SP = 10240
NC = 2
NS = 16
NW = NC * NS
RPW = N // NW
C = 400
NCH = RPW // C
ZR = SP // NS

_mesh = plsc.VectorSubcoreMesh(core_axis_name="c", subcore_axis_name="s")


@functools.partial(
    pl.kernel,
    out_type=(
        jax.ShapeDtypeStruct((NC, SP, D), jnp.float32),
        jax.ShapeDtypeStruct((NC, SP), jnp.float32),
    ),
    mesh=_mesh,
    scratch_types=[
        pltpu.VMEM((C, D), jnp.float32),
        pltpu.VMEM((C,), jnp.int32),
        pltpu.VMEM((C,), jnp.float32),
        pltpu.VMEM_SHARED((SP, D), jnp.float32),
        pltpu.VMEM_SHARED((SP,), jnp.float32),
    ],
)
def _segment_sums(x_hbm, idx_hbm, z2d_hbm, z1d_hbm, sums_out, cnts_out,
                  xbuf, idxbuf, onesbuf, sums_sh, cnts_sh):
    c = lax.axis_index("c")
    s = lax.axis_index("s")
    base = (c * NS + s) * RPW

    pltpu.sync_copy(z2d_hbm.at[pl.ds(s * ZR, ZR)], sums_sh.at[pl.ds(s * ZR, ZR)])
    pltpu.sync_copy(z1d_hbm.at[pl.ds(s * ZR, ZR)], cnts_sh.at[pl.ds(s * ZR, ZR)])

    def set_ones(i, carry):
        onesbuf[pl.ds(i * 16, 16)] = jnp.full((16,), 1.0, jnp.float32)
        return carry

    lax.fori_loop(0, C // 16, set_ones, 0)
    plsc.subcore_barrier()

    def chunk(g, carry):
        off = base + g * C
        pltpu.sync_copy(x_hbm.at[pl.ds(off, C)], xbuf)
        pltpu.sync_copy(idx_hbm.at[pl.ds(off, C)], idxbuf)
        pltpu.sync_copy(xbuf, sums_sh.at[idxbuf], add=True)
        pltpu.sync_copy(onesbuf, cnts_sh.at[idxbuf], add=True)
        return carry

    lax.fori_loop(0, NCH, chunk, 0)
    plsc.subcore_barrier()

    pltpu.sync_copy(sums_sh.at[pl.ds(s * ZR, ZR)], sums_out.at[c, pl.ds(s * ZR, ZR)])
    pltpu.sync_copy(cnts_sh.at[pl.ds(s * ZR, ZR)], cnts_out.at[c, pl.ds(s * ZR, ZR)])


def _combine_body(sums_ref, cnts_ref, mean_ref):
    total = sums_ref[0] + sums_ref[1]
    cnt = cnts_ref[0] + cnts_ref[1]
    mean_ref[...] = total / jnp.maximum(cnt, 1.0)[:, None]


def _combine(sums, cnts):
    return pl.pallas_call(
        _combine_body,
        out_shape=jax.ShapeDtypeStruct((SP, D), jnp.float32),
    )(sums, cnts)


@functools.partial(
    pl.kernel,
    out_type=jax.ShapeDtypeStruct((N, D), jnp.float32),
    mesh=_mesh,
    scratch_types=[
        pltpu.VMEM((C, D), jnp.float32),
        pltpu.VMEM((C, D), jnp.float32),
        pltpu.VMEM((C,), jnp.int32),
        pltpu.SemaphoreType.DMA,
    ],
)
def _subtract(x_hbm, idx_hbm, mean_hbm, out_hbm, xbuf, gbuf, idxbuf, sem):
    c = lax.axis_index("c")
    s = lax.axis_index("s")
    base = (c * NS + s) * RPW

    def chunk(g, carry):
        off = base + g * C
        pltpu.sync_copy(idx_hbm.at[pl.ds(off, C)], idxbuf)
        gat = pltpu.async_copy(mean_hbm.at[idxbuf], gbuf, sem)
        pltpu.sync_copy(x_hbm.at[pl.ds(off, C)], xbuf)
        gat.wait()

        def row(r, rcarry):
            for j in range(D // 16):
                sl = pl.ds(j * 16, 16)
                xbuf[r, sl] = xbuf[r, sl] - gbuf[r, sl]
            return rcarry

        lax.fori_loop(0, C, row, 0)
        pltpu.sync_copy(xbuf, out_hbm.at[pl.ds(off, C)])
        return carry

    lax.fori_loop(0, NCH, chunk, 0)


@jax.jit
def kernel(x, index):
    idx = index.astype(jnp.int32)
    z2d = jnp.zeros((SP, D), jnp.float32)
    z1d = jnp.zeros((SP,), jnp.float32)
    sums, cnts = _segment_sums(x, idx, z2d, z1d)
    mean = _combine(sums, cnts)
    return _subtract(x, idx, mean)


# speedup vs baseline: 2.8267x; 2.8267x over previous
import functools

import jax
import jax.numpy as jnp
from jax import lax
from jax.experimental import pallas as pl
from jax.experimental.pallas import tpu as pltpu
from jax.experimental.pallas import tpu_sc as plsc

N = 320000
D = 128
S = 10000
SP = 10240
NC = 2
NS = 16
NW = NC * NS
RPW = N // NW
C1 = 80
NCH1 = RPW // C1
C3 = 400
NCH3 = RPW // C3
ZR = SP // NS
SEGW = SP // NW
IDXW = N // NS
BITS = [1 << b for b in range(18, -1, -1)]

_mesh = plsc.VectorSubcoreMesh(core_axis_name="c", subcore_axis_name="s")


@functools.partial(
    pl.kernel,
    out_type=(
        jax.ShapeDtypeStruct((NC, SP, D), jnp.float32),
        jax.ShapeDtypeStruct((NW, SEGW), jnp.float32),
    ),
    mesh=_mesh,
    scratch_types=[
        pltpu.VMEM((C1, D), jnp.float32),
        pltpu.VMEM((C1,), jnp.int32),
        pltpu.VMEM((16,), jnp.int32),
        pltpu.VMEM((336,), jnp.int32),
        pltpu.VMEM((SEGW,), jnp.float32),
        pltpu.VMEM((2000,), jnp.int32),
        pltpu.VMEM_SHARED((SP, D), jnp.float32),
        pltpu.VMEM_SHARED((N,), jnp.int32),
        pltpu.SemaphoreType.DMA,
    ],
)
def _segment_sums(x_hbm, idx_hbm, z2d_hbm, sums_out, cnts_out,
                  xbuf, idxbuf, gbuf, lobuf, cntbuf, ibounce, sums_sh, idx_sp,
                  sem):
    c = lax.axis_index("c")
    s = lax.axis_index("s")
    wid = c * NS + s
    base = wid * RPW

    pltpu.sync_copy(z2d_hbm.at[pl.ds(s * ZR, ZR)], sums_sh.at[pl.ds(s * ZR, ZR)])
    def stage(k, carry):
        off = s * IDXW + k * 2000
        pltpu.sync_copy(idx_hbm.at[pl.ds(off, 2000)], ibounce)
        pltpu.sync_copy(ibounce, idx_sp.at[pl.ds(off, 2000)])
        return carry

    lax.fori_loop(0, IDXW // 2000, stage, 0)
    plsc.subcore_barrier()

    def chunk(g, carry):
        off = base + g * C1
        pltpu.sync_copy(x_hbm.at[pl.ds(off, C1)], xbuf)
        pltpu.sync_copy(idx_hbm.at[pl.ds(off, C1)], idxbuf)
        pltpu.sync_copy(xbuf, sums_sh.at[idxbuf], add=True)
        return carry

    lax.fori_loop(0, NCH1, chunk, 0)

    seg0 = wid * SEGW
    lanes = lax.iota(jnp.int32, 16)

    def lb_group(g, carry):
        targets = seg0 + g * 16 + lanes
        ans = jnp.zeros((16,), jnp.int32)
        for bit in BITS:
            t_try = ans + bit
            addr = jnp.minimum(t_try, N) - 1
            pltpu.async_copy(idx_sp.at[addr], gbuf, sem).wait()
            v = gbuf[pl.ds(0, 16)]
            ans = jnp.where((t_try <= N) & (v < targets), t_try, ans)
        lobuf[pl.ds(g * 16, 16)] = ans
        return carry

    lax.fori_loop(0, SEGW // 16 + 1, lb_group, 0)

    def cnt_group(g, carry):
        a = lobuf[pl.ds(g * 16, 16)]
        b = lobuf[pl.ds(g * 16 + 1, 16)]
        cntbuf[pl.ds(g * 16, 16)] = (b - a).astype(jnp.float32)
        return carry

    lax.fori_loop(0, SEGW // 16, cnt_group, 0)
    pltpu.sync_copy(cntbuf, cnts_out.at[wid])

    plsc.subcore_barrier()
    pltpu.sync_copy(sums_sh.at[pl.ds(s * ZR, ZR)], sums_out.at[c, pl.ds(s * ZR, ZR)])


def _combine_body(sums_ref, cnts_ref, mean_ref):
    total = sums_ref[0] + sums_ref[1]
    cnt = cnts_ref[...]
    mean_ref[...] = total / jnp.maximum(cnt, 1.0)[:, None]


def _combine(sums, cnts):
    return pl.pallas_call(
        _combine_body,
        out_shape=jax.ShapeDtypeStruct((SP, D), jnp.float32),
    )(sums, cnts)


@functools.partial(
    pl.kernel,
    out_type=jax.ShapeDtypeStruct((N, D), jnp.float32),
    mesh=_mesh,
    scratch_types=[
        pltpu.VMEM((C3, D), jnp.float32),
        pltpu.VMEM((C3, D), jnp.float32),
        pltpu.VMEM((C3,), jnp.int32),
        pltpu.SemaphoreType.DMA,
    ],
)
def _subtract(x_hbm, idx_hbm, mean_hbm, out_hbm, xbuf, gbuf, idxbuf, sem):
    c = lax.axis_index("c")
    s = lax.axis_index("s")
    base = (c * NS + s) * RPW

    def chunk(g, carry):
        off = base + g * C3
        pltpu.sync_copy(idx_hbm.at[pl.ds(off, C3)], idxbuf)
        gat = pltpu.async_copy(mean_hbm.at[idxbuf], gbuf, sem)
        pltpu.sync_copy(x_hbm.at[pl.ds(off, C3)], xbuf)
        gat.wait()

        def row(r, rcarry):
            for j in range(D // 16):
                sl = pl.ds(j * 16, 16)
                xbuf[r, sl] = xbuf[r, sl] - gbuf[r, sl]
            return rcarry

        lax.fori_loop(0, C3, row, 0)
        pltpu.sync_copy(xbuf, out_hbm.at[pl.ds(off, C3)])
        return carry

    lax.fori_loop(0, NCH3, chunk, 0)


@jax.jit
def kernel(x, index):
    idx = index.astype(jnp.int32)
    z2d = jnp.zeros((SP, D), jnp.float32)
    sums, cnts = _segment_sums(x, idx, z2d)
    mean = _combine(sums, cnts.reshape(SP))
    return _subtract(x, idx, mean)


# speedup vs baseline: 3.2966x; 1.1663x over previous
import functools

import jax
import jax.numpy as jnp
from jax import lax
from jax.experimental import pallas as pl
from jax.experimental.pallas import tpu as pltpu
from jax.experimental.pallas import tpu_sc as plsc

N = 320000
D = 128
S = 10000
SP = 10240
NC = 2
NS = 16
NW = NC * NS
RPW = N // NW
C1 = 80
NCH1 = RPW // C1
C3 = 80
NCH3 = RPW // C3
ZR = SP // NS
SEGW = SP // NW
IDXW = N // NS
BITS = [1 << b for b in range(18, -1, -1)]

_mesh = plsc.VectorSubcoreMesh(core_axis_name="c", subcore_axis_name="s")


@functools.partial(
    pl.kernel,
    out_type=(
        jax.ShapeDtypeStruct((NC, SP, D), jnp.float32),
        jax.ShapeDtypeStruct((NW, SEGW), jnp.float32),
    ),
    mesh=_mesh,
    scratch_types=[
        pltpu.VMEM((C1, D), jnp.float32),
        pltpu.VMEM((C1, D), jnp.float32),
        pltpu.VMEM((C1,), jnp.int32),
        pltpu.VMEM((C1,), jnp.int32),
        pltpu.VMEM((16,), jnp.int32),
        pltpu.VMEM((336,), jnp.int32),
        pltpu.VMEM((SEGW,), jnp.float32),
        pltpu.VMEM((2000,), jnp.int32),
        pltpu.VMEM_SHARED((SP, D), jnp.float32),
        pltpu.VMEM_SHARED((N,), jnp.int32),
        pltpu.SemaphoreType.DMA,
        pltpu.SemaphoreType.DMA,
        pltpu.SemaphoreType.DMA,
    ],
)
def _segment_sums(x_hbm, idx_hbm, z2d_hbm, sums_out, cnts_out,
                  xbuf0, xbuf1, idxbuf0, idxbuf1, gbuf, lobuf, cntbuf, ibounce,
                  sums_sh, idx_sp, sem, semx0, semx1):
    c = lax.axis_index("c")
    s = lax.axis_index("s")
    wid = c * NS + s
    base = wid * RPW

    pltpu.sync_copy(z2d_hbm.at[pl.ds(s * ZR, ZR)], sums_sh.at[pl.ds(s * ZR, ZR)])
    def stage(k, carry):
        off = s * IDXW + k * 2000
        pltpu.sync_copy(idx_hbm.at[pl.ds(off, 2000)], ibounce)
        pltpu.sync_copy(ibounce, idx_sp.at[pl.ds(off, 2000)])
        return carry

    lax.fori_loop(0, IDXW // 2000, stage, 0)
    plsc.subcore_barrier()

    xbufs = (xbuf0, xbuf1)
    idxbufs = (idxbuf0, idxbuf1)
    semxs = (semx0, semx1)
    for b in (0, 1):
        off = base + b * C1
        pltpu.sync_copy(idx_hbm.at[pl.ds(off, C1)], idxbufs[b])
        pltpu.async_copy(x_hbm.at[pl.ds(off, C1)], xbufs[b], semxs[b])

    def chunk(g2, carry):
        for b in (0, 1):
            g = g2 * 2 + b
            off = base + g * C1
            pltpu.make_async_copy(x_hbm.at[pl.ds(off, C1)], xbufs[b],
                                  semxs[b]).wait()
            pltpu.sync_copy(xbufs[b], sums_sh.at[idxbufs[b]], add=True)

            @pl.when(g + 2 < NCH1)
            def _():
                off2 = base + (g + 2) * C1
                pltpu.sync_copy(idx_hbm.at[pl.ds(off2, C1)], idxbufs[b])
                pltpu.async_copy(x_hbm.at[pl.ds(off2, C1)], xbufs[b], semxs[b])
        return carry

    lax.fori_loop(0, NCH1 // 2, chunk, 0)
    off_t = base + (NCH1 - 1) * C1
    pltpu.make_async_copy(x_hbm.at[pl.ds(off_t, C1)], xbufs[0], semxs[0]).wait()
    pltpu.sync_copy(xbufs[0], sums_sh.at[idxbufs[0]], add=True)

    seg0 = wid * SEGW
    lanes = lax.iota(jnp.int32, 16)

    def lb_group(g, carry):
        targets = seg0 + g * 16 + lanes
        ans = jnp.zeros((16,), jnp.int32)
        for bit in BITS:
            t_try = ans + bit
            addr = jnp.minimum(t_try, N) - 1
            pltpu.async_copy(idx_sp.at[addr], gbuf, sem).wait()
            v = gbuf[pl.ds(0, 16)]
            ans = jnp.where((t_try <= N) & (v < targets), t_try, ans)
        lobuf[pl.ds(g * 16, 16)] = ans
        return carry

    lax.fori_loop(0, SEGW // 16 + 1, lb_group, 0)

    def cnt_group(g, carry):
        a = lobuf[pl.ds(g * 16, 16)]
        b = lobuf[pl.ds(g * 16 + 1, 16)]
        cntbuf[pl.ds(g * 16, 16)] = (b - a).astype(jnp.float32)
        return carry

    lax.fori_loop(0, SEGW // 16, cnt_group, 0)
    pltpu.sync_copy(cntbuf, cnts_out.at[wid])

    plsc.subcore_barrier()
    pltpu.sync_copy(sums_sh.at[pl.ds(s * ZR, ZR)], sums_out.at[c, pl.ds(s * ZR, ZR)])


def _combine_body(sums_ref, cnts_ref, mean_ref):
    total = sums_ref[0] + sums_ref[1]
    cnt = cnts_ref[...]
    mean_ref[...] = total / jnp.maximum(cnt, 1.0)[:, None]


def _combine(sums, cnts):
    return pl.pallas_call(
        _combine_body,
        out_shape=jax.ShapeDtypeStruct((SP, D), jnp.float32),
    )(sums, cnts)


@functools.partial(
    pl.kernel,
    out_type=jax.ShapeDtypeStruct((N, D), jnp.float32),
    mesh=_mesh,
    scratch_types=[
        pltpu.VMEM((C3, D), jnp.float32),
        pltpu.VMEM((C3, D), jnp.float32),
        pltpu.VMEM((C3, D), jnp.float32),
        pltpu.VMEM((C3, D), jnp.float32),
        pltpu.VMEM((C3, D), jnp.float32),
        pltpu.VMEM((C3, D), jnp.float32),
        pltpu.VMEM((C3,), jnp.int32),
        pltpu.VMEM((C3,), jnp.int32),
        pltpu.SemaphoreType.DMA,
        pltpu.SemaphoreType.DMA,
        pltpu.SemaphoreType.DMA,
        pltpu.SemaphoreType.DMA,
        pltpu.SemaphoreType.DMA,
        pltpu.SemaphoreType.DMA,
    ],
)
def _subtract(x_hbm, idx_hbm, mean_hbm, out_hbm, xbuf0, xbuf1, gbuf0, gbuf1,
              obuf0, obuf1, idxbuf0, idxbuf1, semx0, semx1, semg0, semg1,
              semo0, semo1):
    c = lax.axis_index("c")
    s = lax.axis_index("s")
    base = (c * NS + s) * RPW
    xbufs = (xbuf0, xbuf1)
    gbufs = (gbuf0, gbuf1)
    obufs = (obuf0, obuf1)
    idxbufs = (idxbuf0, idxbuf1)
    semxs = (semx0, semx1)
    semgs = (semg0, semg1)
    semos = (semo0, semo1)

    for b in (0, 1):
        off = base + b * C3
        pltpu.sync_copy(idx_hbm.at[pl.ds(off, C3)], idxbufs[b])
        pltpu.async_copy(x_hbm.at[pl.ds(off, C3)], xbufs[b], semxs[b])
        pltpu.async_copy(mean_hbm.at[idxbufs[b]], gbufs[b], semgs[b])

    def chunk(g2, carry):
        for b in (0, 1):
            g = g2 * 2 + b
            off = base + g * C3
            pltpu.make_async_copy(x_hbm.at[pl.ds(off, C3)], xbufs[b],
                                  semxs[b]).wait()
            pltpu.make_async_copy(mean_hbm.at[idxbufs[b]], gbufs[b],
                                  semgs[b]).wait()

            @pl.when(g >= 2)
            def _():
                off_prev = base + (g - 2) * C3
                pltpu.make_async_copy(obufs[b], out_hbm.at[pl.ds(off_prev, C3)],
                                      semos[b]).wait()

            def row(r, rcarry):
                for j in range(D // 16):
                    sl = pl.ds(j * 16, 16)
                    obufs[b][r, sl] = xbufs[b][r, sl] - gbufs[b][r, sl]
                return rcarry

            lax.fori_loop(0, C3, row, 0)
            pltpu.async_copy(obufs[b], out_hbm.at[pl.ds(off, C3)], semos[b])

            @pl.when(g + 2 < NCH3)
            def _():
                off2 = base + (g + 2) * C3
                pltpu.sync_copy(idx_hbm.at[pl.ds(off2, C3)], idxbufs[b])
                pltpu.async_copy(x_hbm.at[pl.ds(off2, C3)], xbufs[b], semxs[b])
                pltpu.async_copy(mean_hbm.at[idxbufs[b]], gbufs[b], semgs[b])
        return carry

    lax.fori_loop(0, NCH3 // 2, chunk, 0)
    g_t = NCH3 - 1
    off_t = base + g_t * C3
    pltpu.make_async_copy(x_hbm.at[pl.ds(off_t, C3)], xbufs[0], semxs[0]).wait()
    pltpu.make_async_copy(mean_hbm.at[idxbufs[0]], gbufs[0], semgs[0]).wait()
    pltpu.make_async_copy(obufs[0], out_hbm.at[pl.ds(base + (g_t - 2) * C3, C3)],
                          semos[0]).wait()

    def row_t(r, rcarry):
        for j in range(D // 16):
            sl = pl.ds(j * 16, 16)
            obufs[0][r, sl] = xbufs[0][r, sl] - gbufs[0][r, sl]
        return rcarry

    lax.fori_loop(0, C3, row_t, 0)
    pltpu.async_copy(obufs[0], out_hbm.at[pl.ds(off_t, C3)], semos[0])
    pltpu.make_async_copy(obufs[0], out_hbm.at[pl.ds(off_t, C3)], semos[0]).wait()
    pltpu.make_async_copy(obufs[1], out_hbm.at[pl.ds(base + (g_t - 1) * C3, C3)],
                          semos[1]).wait()


@jax.jit
def kernel(x, index):
    idx = index.astype(jnp.int32)
    z2d = jnp.zeros((SP, D), jnp.float32)
    sums, cnts = _segment_sums(x, idx, z2d)
    mean = _combine(sums, cnts.reshape(SP))
    return _subtract(x, idx, mean)
